# baseline (speedup 1.0000x reference)
import functools

import jax
import jax.numpy as jnp
from jax import lax
from jax.experimental import pallas as pl
from jax.experimental.pallas import tpu as pltpu, tpu_sc as plsc

ALPHA = 0.25


def _w_body(p_ref, w_ref):
    p = p_ref[0]
    neg_cost = (1.0 - ALPHA) * (p * p) * -jnp.log(1.0 - p + 1e-08)
    one_m_p = 1.0 - p
    pos_cost = ALPHA * (one_m_p * one_m_p) * -jnp.log(p + 1e-08)
    w_ref[0] = pos_cost - neg_cost


def _merge_body(mv_ref, mi_ref, t_ref, amin_ref, cid_ref):
    b = pl.program_id(0)
    mv = mv_ref[0]
    mi = mi_ref[0]
    best = jnp.min(mv, axis=1, keepdims=True)
    cand = jnp.where(mv == best, mi, jnp.int32(2 ** 30))
    amin_ref[0, 0, :] = jnp.min(cand, axis=1)
    tt = (t_ref[0] == 1.0).astype(jnp.float32)
    M, C = tt.shape
    tmax = jnp.max(tt, axis=1, keepdims=True)
    ciota = lax.broadcasted_iota(jnp.int32, (M, C), 1)
    cid = jnp.min(jnp.where(tt == tmax, ciota, jnp.int32(C)), axis=1)
    cid_ref[0, 0, :] = cid


def _sc_stage(w, loc_pred_t, cls_true_bin, loc_true):
    B, C, N = w.shape
    M = cls_true_bin.shape[1]
    info = plsc.get_sparse_core_info()
    NC, NS, L = info.num_cores, info.num_subcores, info.num_lanes
    NW = NC * NS
    CH = N // NW
    NJ = CH // L

    mesh = plsc.VectorSubcoreMesh(core_axis_name="c", subcore_axis_name="s")

    @functools.partial(
        pl.kernel, mesh=mesh,
        out_type=[
            jax.ShapeDtypeStruct((NW, B, M * L), jnp.float32),
            jax.ShapeDtypeStruct((NW, B, M * L), jnp.int32),
        ],
        scratch_types=[
            pltpu.VMEM((C, CH), jnp.float32),
            pltpu.VMEM((4, CH), jnp.float32),
            pltpu.VMEM((M * C,), jnp.float32),
            pltpu.VMEM((M * 4,), jnp.float32),
            pltpu.VMEM((L,), jnp.float32),
            pltpu.VMEM((L,), jnp.int32),
            pltpu.VMEM((M * L,), jnp.float32),
            pltpu.VMEM((M * L,), jnp.int32),
        ],
    )
    def sc_kernel(w_hbm, lp_hbm, t_hbm, lt_hbm, mv_hbm, mi_hbm,
                  w_v, lp_v, t_vs, lt_vs, run_v, runj_v, res_v, resi_v):
        wid = lax.axis_index("s") * NC + lax.axis_index("c")
        n0 = wid * CH
        lane = lax.broadcasted_iota(jnp.int32, (L,), 0)
        for b in range(B):
            pltpu.sync_copy(w_hbm.at[b, :, pl.ds(n0, CH)], w_v)
            pltpu.sync_copy(lp_hbm.at[b, :, pl.ds(n0, CH)], lp_v)
            pltpu.sync_copy(t_hbm.at[b], t_vs)
            pltpu.sync_copy(lt_hbm.at[b], lt_vs)

            def m_loop(m):
                run_v[...] = jnp.full((L,), jnp.inf, jnp.float32)
                runj_v[...] = jnp.zeros((L,), jnp.int32)
                def _splat(ref, i):
                    return jnp.broadcast_to(ref[pl.ds(i, 1)], (L,))
                t_ymin = _splat(lt_vs, m * 4 + 0)
                t_xmin = _splat(lt_vs, m * 4 + 1)
                t_ymax = _splat(lt_vs, m * 4 + 2)
                t_xmax = _splat(lt_vs, m * 4 + 3)
                tw = [_splat(t_vs, m * C + c) for c in range(C)]
                te_y = jnp.maximum(t_ymax - t_ymin, 0.0)
                te_x = jnp.maximum(t_xmax - t_xmin, 0.0)
                b2_area = te_y * te_x

                def j_loop(j):
                    o = j * L
                    cls = tw[0] * w_v[0, pl.ds(o, L)]
                    for c in range(1, C):
                        cls = cls + tw[c] * w_v[c, pl.ds(o, L)]
                    p_ymin = lp_v[0, pl.ds(o, L)]
                    p_xmin = lp_v[1, pl.ds(o, L)]
                    p_ymax = lp_v[2, pl.ds(o, L)]
                    p_xmax = lp_v[3, pl.ds(o, L)]
                    reg = (((jnp.abs(t_ymin - p_ymin) + jnp.abs(t_xmin - p_xmin))
                            + jnp.abs(t_ymax - p_ymax)) + jnp.abs(t_xmax - p_xmax))
                    pe_y = jnp.maximum(p_ymax - p_ymin, 0.0)
                    pe_x = jnp.maximum(p_xmax - p_xmin, 0.0)
                    b1_area = pe_y * pe_x
                    d_y = jnp.minimum(p_ymax, t_ymax) - jnp.maximum(p_ymin, t_ymin)
                    d_x = jnp.minimum(p_xmax, t_xmax) - jnp.maximum(p_xmin, t_xmin)
                    inter = jnp.maximum(d_y, 0.0) * jnp.maximum(d_x, 0.0)
                    union = b1_area + b2_area - inter
                    iou = inter / jnp.where(union > 0.0, union, 1.0)
                    enc = ((pe_y + te_y) - d_y) * ((pe_x + te_x) - d_x)
                    corr = (enc - union) / jnp.where(enc > 0.0, enc, 1.0)
                    total = ((cls + 2.5 * reg) + (1.0 - iou)) + corr
                    rv = run_v[...]
                    better = total < rv
                    run_v[...] = jnp.minimum(total, rv)
                    runj_v[...] = jnp.where(better, j, runj_v[...])

                pl.loop(0, NJ)(j_loop)
                res_v[pl.ds(m * L, L)] = run_v[...]
                resi_v[pl.ds(m * L, L)] = (runj_v[...] * L + lane) + n0

            pl.loop(0, M)(m_loop)
            pltpu.sync_copy(res_v, mv_hbm.at[wid, b])
            pltpu.sync_copy(resi_v, mi_hbm.at[wid, b])

    return sc_kernel


def kernel(cls_pred, loc_pred, cls_true, loc_true, reg_mask):
    B, N, C = cls_pred.shape
    M = cls_true.shape[1]

    cls_pred_t = jnp.transpose(cls_pred, (0, 2, 1))
    loc_pred_t = jnp.transpose(loc_pred, (0, 2, 1))
    t_bin = (cls_true == 1.0).astype(jnp.float32)

    w = pl.pallas_call(
        _w_body,
        grid=(B,),
        in_specs=[pl.BlockSpec((1, C, N), lambda b: (b, 0, 0))],
        out_specs=pl.BlockSpec((1, C, N), lambda b: (b, 0, 0)),
        out_shape=jax.ShapeDtypeStruct((B, C, N), jnp.float32),
    )(cls_pred_t)

    t_bin_flat = t_bin.reshape(B, M * C)
    loc_true_flat = loc_true.reshape(B, M * 4)
    mv, mi = _sc_stage(w, loc_pred_t, t_bin, loc_true)(
        w, loc_pred_t, t_bin_flat, loc_true_flat)

    NW = mv.shape[0]
    L = mv.shape[2] // M
    mv_t = jnp.transpose(mv.reshape(NW, B, M, L), (1, 2, 0, 3)).reshape(B, M, NW * L)
    mi_t = jnp.transpose(mi.reshape(NW, B, M, L), (1, 2, 0, 3)).reshape(B, M, NW * L)

    amin, cid = pl.pallas_call(
        _merge_body,
        grid=(B,),
        in_specs=[
            pl.BlockSpec((1, M, NW * L), lambda b: (b, 0, 0)),
            pl.BlockSpec((1, M, NW * L), lambda b: (b, 0, 0)),
            pl.BlockSpec((1, M, C), lambda b: (b, 0, 0)),
        ],
        out_specs=[
            pl.BlockSpec((1, 1, M), lambda b: (b, 0, 0)),
            pl.BlockSpec((1, 1, M), lambda b: (b, 0, 0)),
        ],
        out_shape=[
            jax.ShapeDtypeStruct((B, 1, M), jnp.int32),
            jax.ShapeDtypeStruct((B, 1, M), jnp.int32),
        ],
    )(mv_t, mi_t, cls_true)

    batch = jnp.tile(jnp.arange(B, dtype=jnp.int32)[:, None], (1, M))
    return jnp.stack([batch, amin[:, 0, :], cid[:, 0, :]], axis=-1)


# speedup vs baseline: 5.6820x; 5.6820x over previous
import functools

import jax
import jax.numpy as jnp
from jax.experimental import pallas as pl
from jax.experimental.pallas import tpu as pltpu

ALPHA = 0.25
BN = 4096


def _matcher_body(p_ref, lp_ref, t_ref, lt_ref, amin_ref, cid_ref,
                  bv_ref, bi_ref, *, num_blocks):
    nb = pl.program_id(1)
    C = p_ref.shape[1]
    bn = p_ref.shape[2]
    M = t_ref.shape[1]

    @pl.when(nb == 0)
    def _init():
        bv_ref[...] = jnp.full((M, bn), jnp.inf, jnp.float32)
        bi_ref[...] = jnp.zeros((M, bn), jnp.int32)

    p = p_ref[0]
    neg_cost = (1.0 - ALPHA) * (p * p) * -jnp.log(1.0 - p + 1e-08)
    one_m_p = 1.0 - p
    pos_cost = ALPHA * (one_m_p * one_m_p) * -jnp.log(p + 1e-08)
    w = pos_cost - neg_cost

    t = (t_ref[0] == 1.0).astype(jnp.float32)
    cls_loss = jax.lax.dot_general(
        t, w, (((1,), (0,)), ((), ())),
        precision=jax.lax.Precision.HIGHEST)

    lp = lp_ref[0]
    lt = lt_ref[0]
    p_ymin, p_xmin = lp[0:1, :], lp[1:2, :]
    p_ymax, p_xmax = lp[2:3, :], lp[3:4, :]
    t_ymin, t_xmin = lt[:, 0:1], lt[:, 1:2]
    t_ymax, t_xmax = lt[:, 2:3], lt[:, 3:4]

    reg_loss = (((jnp.abs(t_ymin - p_ymin) + jnp.abs(t_xmin - p_xmin))
                 + jnp.abs(t_ymax - p_ymax)) + jnp.abs(t_xmax - p_xmax))

    pe_y = jnp.maximum(p_ymax - p_ymin, 0.0)
    pe_x = jnp.maximum(p_xmax - p_xmin, 0.0)
    te_y = jnp.maximum(t_ymax - t_ymin, 0.0)
    te_x = jnp.maximum(t_xmax - t_xmin, 0.0)
    b1_area = pe_y * pe_x
    b2_area = te_y * te_x
    i_ymin = jnp.maximum(p_ymin, t_ymin)
    i_xmin = jnp.maximum(p_xmin, t_xmin)
    i_ymax = jnp.minimum(p_ymax, t_ymax)
    i_xmax = jnp.minimum(p_xmax, t_xmax)
    d_y = i_ymax - i_ymin
    d_x = i_xmax - i_xmin
    inter = jnp.maximum(d_y, 0.0) * jnp.maximum(d_x, 0.0)
    union = b1_area + b2_area - inter
    iou = inter / jnp.where(union > 0.0, union, 1.0)
    enc = ((pe_y + te_y) - d_y) * ((pe_x + te_x) - d_x)
    giou_corr = (enc - union) / jnp.where(enc > 0.0, enc, 1.0)

    total = ((cls_loss + 2.5 * reg_loss) + (1.0 - iou)) + giou_corr

    bv = bv_ref[...]
    better = total < bv
    bv_ref[...] = jnp.minimum(total, bv)
    bi_ref[...] = jnp.where(better, nb, bi_ref[...])

    @pl.when(nb == num_blocks - 1)
    def _finish():
        bv = bv_ref[...]
        lane = jax.lax.broadcasted_iota(jnp.int32, (M, bn), 1)
        gidx = bi_ref[...] * bn + lane
        mv = jnp.min(bv, axis=1, keepdims=True)
        cand = jnp.where(bv == mv, gidx, jnp.int32(2 ** 30))
        amin_ref[0, :, :] = jnp.min(cand, axis=1, keepdims=True)
        tt = (t_ref[0] == 1.0).astype(jnp.float32)
        tmax = jnp.max(tt, axis=1, keepdims=True)
        ciota = jax.lax.broadcasted_iota(jnp.int32, (M, C), 1)
        cid = jnp.min(jnp.where(tt == tmax, ciota, jnp.int32(C)), axis=1, keepdims=True)
        cid_ref[0, :, :] = cid


def kernel(cls_pred, loc_pred, cls_true, loc_true, reg_mask):
    B, N, C = cls_pred.shape
    M = cls_true.shape[1]
    num_blocks = N // BN

    cls_pred_t = jnp.transpose(cls_pred, (0, 2, 1))
    loc_pred_t = jnp.transpose(loc_pred, (0, 2, 1))

    amin, cid = pl.pallas_call(
        functools.partial(_matcher_body, num_blocks=num_blocks),
        grid=(B, num_blocks),
        in_specs=[
            pl.BlockSpec((1, C, BN), lambda b, nb: (b, 0, nb)),
            pl.BlockSpec((1, 4, BN), lambda b, nb: (b, 0, nb)),
            pl.BlockSpec((1, M, C), lambda b, nb: (b, 0, 0)),
            pl.BlockSpec((1, M, 4), lambda b, nb: (b, 0, 0)),
        ],
        out_specs=[
            pl.BlockSpec((1, M, 1), lambda b, nb: (b, 0, 0)),
            pl.BlockSpec((1, M, 1), lambda b, nb: (b, 0, 0)),
        ],
        out_shape=[
            jax.ShapeDtypeStruct((B, M, 1), jnp.int32),
            jax.ShapeDtypeStruct((B, M, 1), jnp.int32),
        ],
        scratch_shapes=[
            pltpu.VMEM((M, BN), jnp.float32),
            pltpu.VMEM((M, BN), jnp.int32),
        ],
    )(cls_pred_t, loc_pred_t, cls_true, loc_true)

    batch = jnp.tile(jnp.arange(B, dtype=jnp.int32)[:, None], (1, M))
    return jnp.stack([batch, amin[:, :, 0], cid[:, :, 0]], axis=-1)
